# baseline (speedup 1.0000x reference)
import functools

import jax
import jax.numpy as jnp
import numpy as np
from jax import lax
from jax.experimental import pallas as pl
from jax.experimental.pallas import tpu as pltpu
from jax.experimental.pallas import tpu_sc as plsc

D_MODEL = 64
SEQ_LEN = 2048
BATCH = 16

NUM_CORES = 2
NUM_SUBCORES = 16
NW = NUM_CORES * NUM_SUBCORES
ROWS_PER_W = (SEQ_LEN * BATCH) // NW
POS_PER_W = SEQ_LEN // NW
CHUNK = 128
NCHUNK = ROWS_PER_W // CHUNK


def _make_pe_np(max_len, d_model):
    position = np.arange(0, max_len, dtype=np.float32)[:, None]
    div_term = np.exp(
        np.arange(0, d_model, 2).astype(np.float32) * (-np.log(10000.0) / d_model)
    )
    pe = np.zeros((max_len, d_model), dtype=np.float32)
    pe[:, 0::2] = np.sin(position * div_term)
    pe[:, 1::2] = np.cos(position * div_term)
    return pe


_PE = _make_pe_np(SEQ_LEN, D_MODEL)


def _sc_body(x_hbm, w_hbm, pe_hbm, out_hbm, idx_v, rows_v, pe_v, sem):
    wid = lax.axis_index("s") * NUM_CORES + lax.axis_index("c")
    base = wid * ROWS_PER_W

    pltpu.sync_copy(x_hbm.at[wid], idx_v)
    gathers = [
        pltpu.async_copy(
            w_hbm.at[idx_v.at[j]], rows_v.at[pl.ds(j * CHUNK, CHUNK)], sem
        )
        for j in range(NCHUNK)
    ]
    pltpu.sync_copy(pe_hbm.at[pl.ds(wid * POS_PER_W, POS_PER_W)], pe_v)
    for g in gathers:
        g.wait()

    def body(p, carry):
        pe_regs = [pe_v[p, pl.ds(c * 16, 16)] for c in range(D_MODEL // 16)]
        for r in range(BATCH):
            j = p * BATCH + r
            for c in range(D_MODEL // 16):
                rows_v[j, pl.ds(c * 16, 16)] += pe_regs[c]
        return carry

    lax.fori_loop(0, POS_PER_W, body, 0)

    pltpu.sync_copy(rows_v, out_hbm.at[pl.ds(base, ROWS_PER_W)])


@jax.jit
def kernel(x, W):
    x_blocks = x.reshape(NW, NCHUNK, CHUNK)
    mesh = plsc.VectorSubcoreMesh(core_axis_name="c", subcore_axis_name="s")
    run = functools.partial(
        pl.kernel,
        mesh=mesh,
        compiler_params=pltpu.CompilerParams(use_tc_tiling_on_sc=False),
        out_type=jax.ShapeDtypeStruct((SEQ_LEN * BATCH, D_MODEL), jnp.float32),
        scratch_types=[
            pltpu.VMEM((NCHUNK, CHUNK), jnp.int32),
            pltpu.VMEM((ROWS_PER_W, D_MODEL), jnp.float32),
            pltpu.VMEM((POS_PER_W, D_MODEL), jnp.float32),
            pltpu.SemaphoreType.DMA,
        ],
    )(_sc_body)
    out = run(x_blocks, W, jnp.asarray(_PE))
    return out.reshape(SEQ_LEN, BATCH, D_MODEL)
